# Initial kernel scaffold; baseline (speedup 1.0000x reference)
import jax
import jax.numpy as jnp
from jax.experimental import pallas as pl


def kernel(x, src_mask, max_len, pitch_target, energy_target, duration_target, params, pitch_bins, energy_bins):
    raise NotImplementedError("write your pallas kernel here")


# speedup vs baseline: 26.1249x; 26.1249x over previous
import functools

import jax
import jax.numpy as jnp
from jax import lax
from jax.experimental import pallas as pl
from jax.experimental.pallas import tpu as pltpu
from jax.experimental.pallas import tpu_sc as plsc

_B, _T, _H, _F, _NB, _MAXLEN = 16, 512, 256, 256, 256, 2048
_TAUG = _T + 1


def _dot(a, b):
    return lax.dot_general(a, b, (((1,), (0,)), ((), ())),
                           preferred_element_type=jnp.float32)


def _layer_norm(h, g, b):
    m = jnp.mean(h, axis=1, keepdims=True)
    v = jnp.mean((h - m) ** 2, axis=1, keepdims=True)
    return (h - m) / jnp.sqrt(v + 1e-5) * g + b


def _tc_body(xref, ptref, etref, durref, keepref, wref, vref, eref, binsref,
             x3ref, predref, idxref, melref):
    b = pl.program_id(0)
    x = xref[0]
    keep = keepref[0]
    pt = ptref[0]
    et = etref[0]
    dur_l = durref[0]

    def vrow(r):
        return vref[r:r + 1, :]

    def predictor(p, xin):
        W = [wref[(p * 6 + m) * _H:(p * 6 + m + 1) * _H, :] for m in range(6)]
        b1, g1, be1 = vrow(p * 8 + 0), vrow(p * 8 + 1), vrow(p * 8 + 2)
        b2, g2, be2 = vrow(p * 8 + 3), vrow(p * 8 + 4), vrow(p * 8 + 5)
        wl, blr = vrow(p * 8 + 6), vrow(p * 8 + 7)
        z = jnp.zeros((1, _H), jnp.float32)
        h = _dot(jnp.concatenate([z, xin[:-1]], 0), W[0]) \
            + _dot(xin, W[1]) \
            + _dot(jnp.concatenate([xin[1:], z], 0), W[2]) + b1
        h = jnp.maximum(h, 0.0)
        h = _layer_norm(h, g1, be1)
        h2 = _dot(jnp.concatenate([z, h[:-1]], 0), W[3]) \
            + _dot(h, W[4]) \
            + _dot(jnp.concatenate([h[1:], z], 0), W[5]) + b2
        h2 = jnp.maximum(h2, 0.0)
        h2 = _layer_norm(h2, g2, be2)
        out = jnp.sum(h2 * wl, axis=1, keepdims=True) + blr[0:1, 0:1]
        return out * keep

    def bucket_embed(v_s, brow, table):
        idx = jnp.sum((brow < v_s).astype(jnp.int32), axis=1, keepdims=True)
        lanes = lax.broadcasted_iota(jnp.int32, (_T, _NB), 1)
        onehot = (lanes == idx).astype(jnp.float32)
        return _dot(onehot, table)

    dur_col = predictor(0, x)
    pitch_col = predictor(1, x)
    pe = bucket_embed(pt, binsref[0:1, :], eref[0:_NB, :])
    x2 = x + pe
    energy_col = predictor(2, x2)
    ee = bucket_embed(et, binsref[1:2, :], eref[_NB:2 * _NB, :])
    x3 = x2 + ee

    x3ref[0] = jnp.concatenate([x3, jnp.zeros((1, _H), jnp.float32)], axis=0)
    predref[0] = jnp.concatenate(
        [pitch_col, energy_col, dur_col, jnp.zeros((_T, 125), jnp.float32)],
        axis=1)

    jl = lax.broadcasted_iota(jnp.int32, (_T, _T), 1)
    ts = lax.broadcasted_iota(jnp.int32, (_T, _T), 0)
    cum_s = jnp.sum(jnp.where(jl <= ts, dur_l, 0), axis=1, keepdims=True)

    t_out = lax.broadcasted_iota(jnp.int32, (1, _MAXLEN), 1)
    idxo = jnp.sum((cum_s <= t_out).astype(jnp.int32), axis=0, keepdims=True)
    cumlast = cum_s[_T - 1:_T, :]
    idx_g = b * _TAUG + jnp.where(t_out < cumlast, idxo, _T)
    idxref[0] = idx_g
    melref[0] = jnp.broadcast_to(cumlast, (1, 128))


def _tc_call(x, pt3, et3, dur3, keep3, wflat, vflat, eflat, bins):
    return pl.pallas_call(
        _tc_body,
        grid=(_B,),
        in_specs=[
            pl.BlockSpec((1, _T, _H), lambda b: (b, 0, 0)),
            pl.BlockSpec((1, _T, 1), lambda b: (b, 0, 0)),
            pl.BlockSpec((1, _T, 1), lambda b: (b, 0, 0)),
            pl.BlockSpec((1, 1, _T), lambda b: (b, 0, 0)),
            pl.BlockSpec((1, _T, 1), lambda b: (b, 0, 0)),
            pl.BlockSpec((18 * _H, _F), lambda b: (0, 0)),
            pl.BlockSpec((24, _F), lambda b: (0, 0)),
            pl.BlockSpec((2 * _NB, _H), lambda b: (0, 0)),
            pl.BlockSpec((2, _NB), lambda b: (0, 0)),
        ],
        out_specs=[
            pl.BlockSpec((1, _TAUG, _H), lambda b: (b, 0, 0)),
            pl.BlockSpec((1, _T, 128), lambda b: (b, 0, 0)),
            pl.BlockSpec((1, 1, _MAXLEN), lambda b: (b, 0, 0)),
            pl.BlockSpec((1, 1, 128), lambda b: (b, 0, 0)),
        ],
        out_shape=[
            jax.ShapeDtypeStruct((_B, _TAUG, _H), jnp.float32),
            jax.ShapeDtypeStruct((_B, _T, 128), jnp.float32),
            jax.ShapeDtypeStruct((_B, 1, _MAXLEN), jnp.int32),
            jax.ShapeDtypeStruct((_B, 1, 128), jnp.int32),
        ],
    )(x, pt3, et3, dur3, keep3, wflat, vflat, eflat, bins)


_NC, _NS = 2, 16
_NW = _NC * _NS
_ROWS = _B * _MAXLEN
_RPW = _ROWS // _NW
_CH = 128
_NCHUNK = _RPW // _CH

@functools.cache
def _make_sc_gather():
    mesh = plsc.VectorSubcoreMesh(core_axis_name="c", subcore_axis_name="s",
                                  num_cores=_NC, num_subcores=_NS)

    @functools.partial(
        pl.kernel,
        mesh=mesh,
        out_type=jax.ShapeDtypeStruct((_ROWS, _H), jnp.float32),
        scratch_types=[
            pltpu.VMEM((_CH,), jnp.int32),
            pltpu.VMEM((_CH, _H), jnp.float32),
            pltpu.SemaphoreType.DMA,
        ],
    )
    def sc_gather(xaug, idx, out, idx_v, rows_v, sem):
        wid = lax.axis_index("s") * _NC + lax.axis_index("c")
        base = wid * _RPW
        for k in range(_NCHUNK):
            off = base + k * _CH
            pltpu.sync_copy(idx.at[pl.ds(off, _CH)], idx_v)
            pltpu.async_copy(xaug.at[idx_v], rows_v, sem).wait()
            pltpu.sync_copy(rows_v, out.at[pl.ds(off, _CH)])

    return sc_gather


def _sc_gather(xaug, idx):
    return _make_sc_gather()(xaug, idx)


def kernel(x, src_mask, max_len, pitch_target, energy_target, duration_target,
           params, pitch_bins, energy_bins):
    preds = (params['dur'], params['pitch'], params['energy'])
    wflat = jnp.concatenate(
        [p[wn][:, :, k].T for p in preds for wn in ('W1', 'W2')
         for k in range(3)], axis=0)
    vflat = jnp.stack(
        [r for p in preds
         for r in (p['b1'], p['g1'], p['be1'], p['b2'], p['g2'], p['be2'],
                   p['Wl'][0], jnp.broadcast_to(p['bl'], (_F,)))], axis=0)
    eflat = jnp.concatenate([params['pitch_emb'], params['energy_emb']], 0)
    inf = jnp.full((1,), jnp.inf, jnp.float32)
    bins = jnp.stack([jnp.concatenate([pitch_bins.astype(jnp.float32), inf]),
                      jnp.concatenate([energy_bins.astype(jnp.float32), inf])])

    keep3 = (1.0 - src_mask.astype(jnp.float32)).reshape(_B, _T, 1)
    pt3 = pitch_target.reshape(_B, _T, 1)
    et3 = energy_target.reshape(_B, _T, 1)
    dur3 = duration_target.astype(jnp.int32).reshape(_B, 1, _T)

    x3a, pcols, idxg, melb = _tc_call(x, pt3, et3, dur3, keep3,
                                      wflat, vflat, eflat, bins)

    out_rows = _sc_gather(x3a.reshape(_B * _TAUG, _H),
                          idxg.reshape(_ROWS))
    out = out_rows.reshape(_B, _MAXLEN, _H)

    pitch_prediction = pcols[:, :, 0]
    energy_prediction = pcols[:, :, 1]
    log_duration_prediction = pcols[:, :, 2]
    mel_len = jnp.minimum(melb[:, 0, 0], max_len)
    return (out, pitch_prediction, energy_prediction, log_duration_prediction,
            duration_target, mel_len)


# speedup vs baseline: 34.2669x; 1.3117x over previous
import functools

import jax
import jax.numpy as jnp
from jax import lax
from jax.experimental import pallas as pl
from jax.experimental.pallas import tpu as pltpu
from jax.experimental.pallas import tpu_sc as plsc

_B, _T, _H, _F, _NB, _MAXLEN = 16, 512, 256, 256, 256, 2048
_TAUG = _T + 1


def _dot(a, b):
    return lax.dot_general(a, b, (((1,), (0,)), ((), ())),
                           preferred_element_type=jnp.float32)


def _layer_norm(h, g, b):
    m = jnp.mean(h, axis=1, keepdims=True)
    v = jnp.mean((h - m) ** 2, axis=1, keepdims=True)
    return (h - m) / jnp.sqrt(v + 1e-5) * g + b


def _embed_body(xref, ptref, etref, durref, eref, binsref,
                x2ref, x3ref, idxref, melref):
    b = pl.program_id(0)
    x = xref[0]
    pt = ptref[0]
    et = etref[0]
    dur_l = durref[0]

    def bucket_embed(v_s, brow, table):
        idx = jnp.sum((brow < v_s).astype(jnp.int32), axis=1, keepdims=True)
        lanes = lax.broadcasted_iota(jnp.int32, (_T, _NB), 1)
        onehot = (lanes == idx).astype(jnp.float32)
        return _dot(onehot, table)

    x2 = x + bucket_embed(pt, binsref[0:1, :], eref[0:_NB, :])
    x3 = x2 + bucket_embed(et, binsref[1:2, :], eref[_NB:2 * _NB, :])
    x2ref[0] = x2
    x3ref[0] = jnp.concatenate([x3, jnp.zeros((1, _H), jnp.float32)], axis=0)

    jl = lax.broadcasted_iota(jnp.int32, (_T, _T), 1)
    ts = lax.broadcasted_iota(jnp.int32, (_T, _T), 0)
    cum_s = jnp.sum(jnp.where(jl <= ts, dur_l, 0), axis=1, keepdims=True)

    t_out = lax.broadcasted_iota(jnp.int32, (1, _MAXLEN), 1)
    idxo = jnp.sum((cum_s <= t_out).astype(jnp.int32), axis=0, keepdims=True)
    cumlast = cum_s[_T - 1:_T, :]
    idxref[0] = b * _TAUG + jnp.where(t_out < cumlast, idxo, _T)
    melref[0] = jnp.broadcast_to(cumlast, (1, 128))


def _embed_call(x, pt3, et3, dur3, eflat, bins):
    return pl.pallas_call(
        _embed_body,
        grid=(_B,),
        in_specs=[
            pl.BlockSpec((1, _T, _H), lambda b: (b, 0, 0)),
            pl.BlockSpec((1, _T, 1), lambda b: (b, 0, 0)),
            pl.BlockSpec((1, _T, 1), lambda b: (b, 0, 0)),
            pl.BlockSpec((1, 1, _T), lambda b: (b, 0, 0)),
            pl.BlockSpec((2 * _NB, _H), lambda b: (0, 0)),
            pl.BlockSpec((2, _NB), lambda b: (0, 0)),
        ],
        out_specs=[
            pl.BlockSpec((1, _T, _H), lambda b: (b, 0, 0)),
            pl.BlockSpec((1, _TAUG, _H), lambda b: (b, 0, 0)),
            pl.BlockSpec((1, 1, _MAXLEN), lambda b: (b, 0, 0)),
            pl.BlockSpec((1, 1, 128), lambda b: (b, 0, 0)),
        ],
        out_shape=[
            jax.ShapeDtypeStruct((_B, _T, _H), jnp.float32),
            jax.ShapeDtypeStruct((_B, _TAUG, _H), jnp.float32),
            jax.ShapeDtypeStruct((_B, 1, _MAXLEN), jnp.int32),
            jax.ShapeDtypeStruct((_B, 1, 128), jnp.int32),
        ],
    )(x, pt3, et3, dur3, eflat, bins)


def _pred_body(xref, x2ref, keepref, wref, vref, predref):
    x = xref[0]
    x2 = x2ref[0]
    keep = keepref[0]

    def vrow(r):
        return vref[r:r + 1, :]

    def shift_cat(h):
        z = jnp.zeros((1, _H), h.dtype)
        return jnp.concatenate(
            [jnp.concatenate([z, h[:-1]], 0), h,
             jnp.concatenate([h[1:], z], 0)], axis=1)

    def predictor(p, xin):
        W1 = wref[p * 6 * _H:(p * 6 + 3) * _H, :]
        W2 = wref[(p * 6 + 3) * _H:(p * 6 + 6) * _H, :]
        b1, g1, be1 = vrow(p * 8 + 0), vrow(p * 8 + 1), vrow(p * 8 + 2)
        b2, g2, be2 = vrow(p * 8 + 3), vrow(p * 8 + 4), vrow(p * 8 + 5)
        wl, blr = vrow(p * 8 + 6), vrow(p * 8 + 7)
        h = _dot(shift_cat(xin.astype(jnp.bfloat16)), W1) + b1
        h = _layer_norm(jnp.maximum(h, 0.0), g1, be1)
        h2 = _dot(shift_cat(h.astype(jnp.bfloat16)), W2) + b2
        h2 = _layer_norm(jnp.maximum(h2, 0.0), g2, be2)
        out = jnp.sum(h2 * wl, axis=1, keepdims=True) + blr[0:1, 0:1]
        return out * keep

    predref[0] = jnp.concatenate(
        [predictor(1, x), predictor(2, x2), predictor(0, x),
         jnp.zeros((_T, 125), jnp.float32)], axis=1)


def _pred_call(x, x2a, keep3, wflat, vflat):
    return pl.pallas_call(
        _pred_body,
        grid=(_B,),
        in_specs=[
            pl.BlockSpec((1, _T, _H), lambda b: (b, 0, 0)),
            pl.BlockSpec((1, _T, _H), lambda b: (b, 0, 0)),
            pl.BlockSpec((1, _T, 1), lambda b: (b, 0, 0)),
            pl.BlockSpec((18 * _H, _F), lambda b: (0, 0)),
            pl.BlockSpec((24, _F), lambda b: (0, 0)),
        ],
        out_specs=[pl.BlockSpec((1, _T, 128), lambda b: (b, 0, 0))],
        out_shape=[jax.ShapeDtypeStruct((_B, _T, 128), jnp.float32)],
    )(x, x2a, keep3, wflat, vflat)[0]


_NC, _NS = 2, 16
_NW = _NC * _NS
_ROWS = _B * _MAXLEN
_RPW = _ROWS // _NW
_CH = 128
_NCHUNK = _RPW // _CH
_NBUF = 3


@functools.cache
def _make_sc_gather():
    mesh = plsc.VectorSubcoreMesh(core_axis_name="c", subcore_axis_name="s",
                                  num_cores=_NC, num_subcores=_NS)

    @functools.partial(
        pl.kernel,
        mesh=mesh,
        out_type=jax.ShapeDtypeStruct((_ROWS, _H), jnp.float32),
        scratch_types=[
            [pltpu.VMEM((_CH,), jnp.int32) for _ in range(_NBUF)],
            [pltpu.VMEM((_CH, _H), jnp.float32) for _ in range(_NBUF)],
            [pltpu.SemaphoreType.DMA for _ in range(_NBUF)],
            [pltpu.SemaphoreType.DMA for _ in range(_NBUF)],
        ],
    )
    def sc_gather(xaug, idx, out, idx_v, rows_v, gsem, ssem):
        wid = lax.axis_index("s") * _NC + lax.axis_index("c")
        base = wid * _RPW
        gh, sh = {}, {}

        def start_gather(k, s):
            pltpu.sync_copy(idx.at[pl.ds(base + k * _CH, _CH)], idx_v[s])
            gh[k] = pltpu.async_copy(xaug.at[idx_v[s]], rows_v[s], gsem[s])

        for k in range(min(_NBUF, _NCHUNK)):
            start_gather(k, k % _NBUF)
        for k in range(_NCHUNK):
            s = k % _NBUF
            gh[k].wait()
            sh[k] = pltpu.async_copy(
                rows_v[s], out.at[pl.ds(base + k * _CH, _CH)], ssem[s])
            if k + _NBUF < _NCHUNK:
                sh[k].wait()
                start_gather(k + _NBUF, s)
        for k in range(max(_NCHUNK - _NBUF, 0), _NCHUNK):
            sh[k].wait()

    return sc_gather


def _sc_gather(xaug, idx):
    return _make_sc_gather()(xaug, idx)


def kernel(x, src_mask, max_len, pitch_target, energy_target, duration_target,
           params, pitch_bins, energy_bins):
    preds = (params['dur'], params['pitch'], params['energy'])
    wflat = jnp.concatenate(
        [p[wn][:, :, k].T for p in preds for wn in ('W1', 'W2')
         for k in range(3)], axis=0).astype(jnp.bfloat16)
    vflat = jnp.stack(
        [r for p in preds
         for r in (p['b1'], p['g1'], p['be1'], p['b2'], p['g2'], p['be2'],
                   p['Wl'][0], jnp.broadcast_to(p['bl'], (_F,)))], axis=0)
    eflat = jnp.concatenate([params['pitch_emb'], params['energy_emb']], 0)
    inf = jnp.full((1,), jnp.inf, jnp.float32)
    bins = jnp.stack([jnp.concatenate([pitch_bins.astype(jnp.float32), inf]),
                      jnp.concatenate([energy_bins.astype(jnp.float32), inf])])

    keep3 = (1.0 - src_mask.astype(jnp.float32)).reshape(_B, _T, 1)
    pt3 = pitch_target.reshape(_B, _T, 1)
    et3 = energy_target.reshape(_B, _T, 1)
    dur3 = duration_target.astype(jnp.int32).reshape(_B, 1, _T)

    x2a, x3a, idxg, melb = _embed_call(x, pt3, et3, dur3, eflat, bins)
    out_rows = _sc_gather(x3a.reshape(_B * _TAUG, _H), idxg.reshape(_ROWS))
    pcols = _pred_call(x, x2a, keep3, wflat, vflat)
    out = out_rows.reshape(_B, _MAXLEN, _H)

    pitch_prediction = pcols[:, :, 0]
    energy_prediction = pcols[:, :, 1]
    log_duration_prediction = pcols[:, :, 2]
    mel_len = jnp.minimum(melb[:, 0, 0], max_len)
    return (out, pitch_prediction, energy_prediction, log_duration_prediction,
            duration_target, mel_len)
